# baseline (speedup 1.0000x reference)
import functools

import jax
import jax.numpy as jnp
from jax import lax
from jax.experimental import pallas as pl
from jax.experimental.pallas import tpu as pltpu
from jax.experimental.pallas import tpu_sc as plsc

LANES = 16
IDX_CHUNK = 128


@functools.lru_cache(maxsize=None)
def _make_sc_kernel(batch, embed, n_user, n_anime):
    info = plsc.get_sparse_core_info()
    num_cores, num_subcores = info.num_cores, info.num_subcores
    num_workers = num_cores * num_subcores
    bpw = batch // num_workers
    n_chunks = bpw // IDX_CHUNK
    n_groups = bpw // LANES

    mesh = plsc.VectorSubcoreMesh(core_axis_name="c", subcore_axis_name="s")

    @functools.partial(
        pl.kernel,
        mesh=mesh,
        out_type=jax.ShapeDtypeStruct((batch,), jnp.float32),
        compiler_params=pltpu.CompilerParams(
            needs_layout_passes=False, use_tc_tiling_on_sc=False),
        scratch_types=[
            pltpu.VMEM((n_chunks, IDX_CHUNK), jnp.int32),
            pltpu.VMEM((n_chunks, IDX_CHUNK), jnp.int32),
            pltpu.VMEM((embed, bpw), jnp.float32),
            pltpu.VMEM((embed, bpw), jnp.float32),
            pltpu.VMEM((LANES,), jnp.float32),
            pltpu.VMEM((LANES,), jnp.float32),
            pltpu.VMEM((bpw,), jnp.float32),
            pltpu.SemaphoreType.DMA,
        ],
    )
    def sc_kernel(uidx_hbm, aidx_hbm, utab_hbm, atab_hbm, w_hbm, b_hbm,
                  out_hbm, uidx_v, aidx_v, uval_v, aval_v, w_v, b_v,
                  out_v, sem):
        wid = lax.axis_index("s") * num_cores + lax.axis_index("c")
        pltpu.sync_copy(uidx_hbm.at[wid], uidx_v)
        pltpu.sync_copy(aidx_hbm.at[wid], aidx_v)
        pltpu.sync_copy(w_hbm, w_v)
        pltpu.sync_copy(b_hbm, b_v)

        copies = []
        for e in range(embed):
            urow = utab_hbm.at[pl.ds(e * n_user, n_user)]
            arow = atab_hbm.at[pl.ds(e * n_anime, n_anime)]
            for j in range(n_chunks):
                cols = pl.ds(j * IDX_CHUNK, IDX_CHUNK)
                copies.append(pltpu.async_copy(
                    urow.at[uidx_v.at[j]], uval_v.at[e, cols], sem))
                copies.append(pltpu.async_copy(
                    arow.at[aidx_v.at[j]], aval_v.at[e, cols], sem))
        for c in copies:
            c.wait()

        wv = w_v[...]
        bv = b_v[...]

        def body(g, carry):
            cols = pl.ds(g * LANES, LANES)
            acc = jnp.zeros((LANES,), jnp.float32)
            for e in range(embed):
                acc = acc + uval_v[e, cols] * aval_v[e, cols]
            s = 1.0 / (1.0 + jnp.exp(-acc))
            y = wv * s + bv
            o = 1.0 / (1.0 + jnp.exp(-y))
            out_v[cols] = o
            return carry

        lax.fori_loop(0, n_groups, body, 0)
        pltpu.sync_copy(out_v, out_hbm.at[pl.ds(wid * bpw, bpw)])

    return sc_kernel


def kernel(user_input, anime_input, user_table, anime_table, dense_w, dense_b):
    batch = user_input.shape[0]
    n_user, embed = user_table.shape
    n_anime = anime_table.shape[0]
    info = plsc.get_sparse_core_info()
    num_workers = info.num_cores * info.num_subcores
    n_chunks = batch // num_workers // IDX_CHUNK

    uidx = user_input.astype(jnp.int32).reshape(num_workers, n_chunks, IDX_CHUNK)
    aidx = anime_input.astype(jnp.int32).reshape(num_workers, n_chunks, IDX_CHUNK)
    utab_flat = jnp.concatenate([user_table[:, e] for e in range(embed)])
    atab_flat = jnp.concatenate([anime_table[:, e] for e in range(embed)])
    w_splat = jnp.full((LANES,), dense_w.reshape(())[()], dtype=jnp.float32)
    b_splat = jnp.full((LANES,), dense_b.reshape(())[()], dtype=jnp.float32)

    sc = _make_sc_kernel(batch, embed, n_user, n_anime)
    out = sc(uidx, aidx, utab_flat, atab_flat, w_splat, b_splat)
    return out.reshape(batch, 1)


# speedup vs baseline: 3.6089x; 3.6089x over previous
import functools

import jax
import jax.numpy as jnp
from jax import lax
from jax.experimental import pallas as pl
from jax.experimental.pallas import tpu as pltpu
from jax.experimental.pallas import tpu_sc as plsc

EMBED = 32
LANES = 16
IDX_CHUNK = 128


@functools.lru_cache(maxsize=None)
def _make_sc_kernel(batch, embed):
    info = plsc.get_sparse_core_info()
    num_cores, num_subcores = info.num_cores, info.num_subcores
    num_workers = num_cores * num_subcores
    bpw = batch // num_workers
    n_chunks = bpw // IDX_CHUNK
    n_groups = bpw // LANES

    mesh = plsc.VectorSubcoreMesh(core_axis_name="c", subcore_axis_name="s")

    @functools.partial(
        pl.kernel,
        mesh=mesh,
        out_type=jax.ShapeDtypeStruct((batch,), jnp.float32),
        compiler_params=pltpu.CompilerParams(
            needs_layout_passes=False, use_tc_tiling_on_sc=False),
        scratch_types=[
            pltpu.VMEM((n_chunks, IDX_CHUNK), jnp.int32),
            pltpu.VMEM((n_chunks, IDX_CHUNK), jnp.int32),
            pltpu.VMEM((bpw, embed), jnp.float32),
            pltpu.VMEM((bpw, embed), jnp.float32),
            pltpu.VMEM((LANES,), jnp.float32),
            pltpu.VMEM((LANES,), jnp.float32),
            pltpu.VMEM((bpw,), jnp.float32),
            pltpu.SemaphoreType.DMA,
        ],
    )
    def sc_kernel(uidx_hbm, aidx_hbm, utab_hbm, atab_hbm, w_hbm, b_hbm,
                  out_hbm, uidx_v, aidx_v, urows_v, arows_v, w_v, b_v,
                  out_v, sem):
        wid = lax.axis_index("s") * num_cores + lax.axis_index("c")
        pltpu.sync_copy(uidx_hbm.at[wid], uidx_v)
        pltpu.sync_copy(aidx_hbm.at[wid], aidx_v)
        pltpu.sync_copy(w_hbm, w_v)
        pltpu.sync_copy(b_hbm, b_v)

        copies = []
        for j in range(n_chunks):
            rows = pl.ds(j * IDX_CHUNK, IDX_CHUNK)
            copies.append(
                pltpu.async_copy(utab_hbm.at[uidx_v.at[j]], urows_v.at[rows], sem))
            copies.append(
                pltpu.async_copy(atab_hbm.at[aidx_v.at[j]], arows_v.at[rows], sem))
        for c in copies:
            c.wait()

        wv = w_v[...]
        bv = b_v[...]
        iota = lax.iota(jnp.int32, LANES)

        def body(g, carry):
            row = g * LANES + iota
            acc = jnp.zeros((LANES,), jnp.float32)
            for e in range(embed):
                ev = jnp.full((LANES,), e, jnp.int32)
                uu = plsc.load_gather(urows_v, [row, ev])
                aa = plsc.load_gather(arows_v, [row, ev])
                acc = acc + uu * aa
            s = 1.0 / (1.0 + jnp.exp(-acc))
            y = wv * s + bv
            o = 1.0 / (1.0 + jnp.exp(-y))
            out_v[pl.ds(g * LANES, LANES)] = o
            return carry

        lax.fori_loop(0, n_groups, body, 0)
        pltpu.sync_copy(out_v, out_hbm.at[pl.ds(wid * bpw, bpw)])

    return sc_kernel


def kernel(user_input, anime_input, user_table, anime_table, dense_w, dense_b):
    batch = user_input.shape[0]
    embed = user_table.shape[1]
    info = plsc.get_sparse_core_info()
    num_workers = info.num_cores * info.num_subcores
    n_chunks = batch // num_workers // IDX_CHUNK

    uidx = user_input.astype(jnp.int32).reshape(num_workers, n_chunks, IDX_CHUNK)
    aidx = anime_input.astype(jnp.int32).reshape(num_workers, n_chunks, IDX_CHUNK)
    w_splat = jnp.full((LANES,), dense_w.reshape(())[()], dtype=jnp.float32)
    b_splat = jnp.full((LANES,), dense_b.reshape(())[()], dtype=jnp.float32)

    sc = _make_sc_kernel(batch, embed)
    out = sc(uidx, aidx, user_table, anime_table, w_splat, b_splat)
    return out.reshape(batch, 1)
